# baseline (speedup 1.0000x reference)
import functools

import jax
import jax.numpy as jnp
from jax import lax
from jax.experimental import pallas as pl
from jax.experimental.pallas import tpu as pltpu
from jax.experimental.pallas import tpu_sc as plsc

_INFO = plsc.get_sparse_core_info()
_NC = _INFO.num_cores
_NS = _INFO.num_subcores
_NW = _NC * _NS
_L = _INFO.num_lanes

_CHUNK = 64


def _table_body(protos_ref, out_ref):
    p = protos_ref.shape[1]
    out_ref[...] = jnp.sum(protos_ref[...], axis=1) * (0.5 / p)


def _make_sc_lookup(batch, dim):
    assert batch % (_NW * _CHUNK) == 0
    bpw = batch // _NW
    steps = bpw // _CHUNK
    mesh = plsc.VectorSubcoreMesh(core_axis_name="c", subcore_axis_name="s")

    @functools.partial(
        pl.kernel,
        out_type=jax.ShapeDtypeStruct((batch, dim), jnp.float32),
        mesh=mesh,
        scratch_types=[
            pltpu.VMEM((_CHUNK,), jnp.int32),
            pltpu.VMEM((_CHUNK, dim), jnp.float32),
            pltpu.VMEM((_CHUNK, dim), jnp.float32),
            pltpu.SemaphoreType.DMA,
            pltpu.SemaphoreType.DMA,
        ],
    )
    def sc_lookup(feat_hbm, stages_hbm, table_hbm, out_hbm, idx_v, feat_v,
                  rows_v, gsem, fsem):
        wid = lax.axis_index("s") * _NC + lax.axis_index("c")
        base = wid * bpw
        for step in range(steps):
            row0 = base + step * _CHUNK
            pltpu.sync_copy(stages_hbm.at[pl.ds(row0, _CHUNK)], idx_v)
            feat_cp = pltpu.async_copy(
                feat_hbm.at[pl.ds(row0, _CHUNK)], feat_v, fsem)
            gather_cp = pltpu.async_copy(table_hbm.at[idx_v], rows_v, gsem)
            feat_cp.wait()
            gather_cp.wait()

            @plsc.parallel_loop(0, _CHUNK, unroll=2)
            def add_row(r):
                for c in range(dim // _L):
                    s = pl.ds(c * _L, _L)
                    plsc.addupdate(feat_v.at[r, s], rows_v[r, s])

            pltpu.sync_copy(feat_v, out_hbm.at[pl.ds(row0, _CHUNK)])

    return sc_lookup


def kernel(features, class_ids, stages, shared_protos):
    del class_ids
    table = pl.pallas_call(
        _table_body,
        out_shape=jax.ShapeDtypeStruct(
            (shared_protos.shape[0], shared_protos.shape[2]), jnp.float32),
    )(shared_protos)
    batch, dim = features.shape
    return _make_sc_lookup(batch, dim)(features, stages, table)


# speedup vs baseline: 1.3710x; 1.3710x over previous
import functools

import jax
import jax.numpy as jnp
from jax import lax
from jax.experimental import pallas as pl
from jax.experimental.pallas import tpu as pltpu
from jax.experimental.pallas import tpu_sc as plsc

_INFO = plsc.get_sparse_core_info()
_NC = _INFO.num_cores
_NS = _INFO.num_subcores
_NW = _NC * _NS
_L = _INFO.num_lanes

_CHUNK = 64


def _table_body(protos_ref, out_ref):
    s, p, d = protos_ref.shape
    means = jnp.sum(protos_ref[...], axis=1) * (0.5 / p)
    out_ref[...] = jnp.broadcast_to(means, (_NW, s, d)).reshape(_NW * s, d)


def _make_sc_lookup(batch, dim):
    assert batch % (_NW * _CHUNK) == 0
    bpw = batch // _NW
    steps = bpw // _CHUNK
    mesh = plsc.VectorSubcoreMesh(core_axis_name="c", subcore_axis_name="s")

    @functools.partial(
        pl.kernel,
        out_type=jax.ShapeDtypeStruct((batch, dim), jnp.float32),
        mesh=mesh,
        scratch_types=[
            pltpu.VMEM((_CHUNK,), jnp.int32),
            pltpu.VMEM((_CHUNK, dim), jnp.float32),
            pltpu.VMEM((_CHUNK, dim), jnp.float32),
            pltpu.SemaphoreType.DMA,
            pltpu.SemaphoreType.DMA,
        ],
    )
    def sc_lookup(feat_hbm, stages_hbm, table_hbm, out_hbm, idx_v, feat_v,
                  rows_v, gsem, fsem):
        wid = lax.axis_index("s") * _NC + lax.axis_index("c")
        base = wid * bpw
        for step in range(steps):
            row0 = base + step * _CHUNK
            pltpu.sync_copy(stages_hbm.at[pl.ds(row0, _CHUNK)], idx_v)
            for c in range(_CHUNK // _L):
                s = pl.ds(c * _L, _L)
                idx_v[s] = idx_v[s] + wid * 8
            feat_cp = pltpu.async_copy(
                feat_hbm.at[pl.ds(row0, _CHUNK)], feat_v, fsem)
            gather_cp = pltpu.async_copy(table_hbm.at[idx_v], rows_v, gsem)
            feat_cp.wait()
            gather_cp.wait()

            @plsc.parallel_loop(0, _CHUNK, unroll=2)
            def add_row(r):
                for c in range(dim // _L):
                    s = pl.ds(c * _L, _L)
                    plsc.addupdate(feat_v.at[r, s], rows_v[r, s])

            pltpu.sync_copy(feat_v, out_hbm.at[pl.ds(row0, _CHUNK)])

    return sc_lookup


def kernel(features, class_ids, stages, shared_protos):
    del class_ids
    table = pl.pallas_call(
        _table_body,
        out_shape=jax.ShapeDtypeStruct(
            (_NW * shared_protos.shape[0], shared_protos.shape[2]),
            jnp.float32),
    )(shared_protos)
    batch, dim = features.shape
    return _make_sc_lookup(batch, dim)(features, stages, table)


# speedup vs baseline: 1.5047x; 1.0975x over previous
import functools

import jax
import jax.numpy as jnp
from jax import lax
from jax.experimental import pallas as pl
from jax.experimental.pallas import tpu as pltpu
from jax.experimental.pallas import tpu_sc as plsc

_INFO = plsc.get_sparse_core_info()
_NC = _INFO.num_cores
_NS = _INFO.num_subcores
_NW = _NC * _NS
_L = _INFO.num_lanes

_CHUNK = 32


def _table_body(protos_ref, out_ref):
    s, p, d = protos_ref.shape
    means = jnp.sum(protos_ref[...], axis=1) * (0.5 / p)
    out_ref[...] = jnp.broadcast_to(means, (_NW, s, d)).reshape(_NW * s, d)


def _make_sc_lookup(batch, dim):
    assert batch % (_NW * _CHUNK) == 0
    bpw = batch // _NW
    steps = bpw // _CHUNK
    mesh = plsc.VectorSubcoreMesh(core_axis_name="c", subcore_axis_name="s")

    @functools.partial(
        pl.kernel,
        out_type=jax.ShapeDtypeStruct((batch, dim), jnp.float32),
        mesh=mesh,
        scratch_types=[
            [pltpu.VMEM((_CHUNK,), jnp.int32)] * 2,
            [pltpu.VMEM((_CHUNK, dim), jnp.float32)] * 2,
            [pltpu.VMEM((_CHUNK, dim), jnp.float32)] * 2,
            [pltpu.SemaphoreType.DMA] * 2,
            [pltpu.SemaphoreType.DMA] * 2,
            [pltpu.SemaphoreType.DMA] * 2,
        ],
    )
    def sc_lookup(feat_hbm, stages_hbm, table_hbm, out_hbm, idx_v, feat_v,
                  rows_v, gsem, fsem, osem):
        wid = lax.axis_index("s") * _NC + lax.axis_index("c")
        base = wid * bpw

        def start_in(step, b):
            row0 = base + step * _CHUNK
            if step >= 2:
                pltpu.make_async_copy(
                    feat_v[b], out_hbm.at[pl.ds(0, _CHUNK)], osem[b]).wait()
            pltpu.sync_copy(stages_hbm.at[pl.ds(row0, _CHUNK)], idx_v[b])
            for c in range(_CHUNK // _L):
                s = pl.ds(c * _L, _L)
                idx_v[b][s] = idx_v[b][s] + wid * 8
            pltpu.async_copy(feat_hbm.at[pl.ds(row0, _CHUNK)], feat_v[b],
                             fsem[b])
            pltpu.async_copy(table_hbm.at[idx_v[b]], rows_v[b], gsem[b])

        def wait_in(b):
            pltpu.make_async_copy(feat_hbm.at[pl.ds(0, _CHUNK)], feat_v[b],
                                  fsem[b]).wait()
            pltpu.make_async_copy(table_hbm.at[idx_v[b]], rows_v[b],
                                  gsem[b]).wait()

        start_in(0, 0)
        for step in range(steps):
            b = step % 2
            if step + 1 < steps:
                start_in(step + 1, 1 - b)
            wait_in(b)

            @plsc.parallel_loop(0, _CHUNK, unroll=2)
            def add_row(r):
                for c in range(dim // _L):
                    s = pl.ds(c * _L, _L)
                    plsc.addupdate(feat_v[b].at[r, s], rows_v[b][r, s])

            row0 = base + step * _CHUNK
            pltpu.async_copy(feat_v[b], out_hbm.at[pl.ds(row0, _CHUNK)],
                             osem[b])
        for b in range(2):
            pltpu.make_async_copy(
                feat_v[b], out_hbm.at[pl.ds(0, _CHUNK)], osem[b]).wait()

    return sc_lookup


def kernel(features, class_ids, stages, shared_protos):
    del class_ids
    table = pl.pallas_call(
        _table_body,
        out_shape=jax.ShapeDtypeStruct(
            (_NW * shared_protos.shape[0], shared_protos.shape[2]),
            jnp.float32),
    )(shared_protos)
    batch, dim = features.shape
    return _make_sc_lookup(batch, dim)(features, stages, table)
